# Initial kernel scaffold; baseline (speedup 1.0000x reference)
import jax
import jax.numpy as jnp
from jax.experimental import pallas as pl


def kernel(x, edge_index, Wl0, bl0, Wr0, gamma0, beta0, Wl1, bl1, Wr1, gamma1, beta1, Wl2, bl2, Wr2, gamma2, beta2):
    raise NotImplementedError("write your pallas kernel here")


# speedup vs baseline: 2.4742x; 2.4742x over previous
import functools

import jax
import jax.numpy as jnp
from jax import lax
from jax.experimental import pallas as pl
from jax.experimental.pallas import tpu as pltpu
from jax.experimental.pallas import tpu_sc as plsc

N = 10000
E = 320000
D = 128
EPS = 1e-5

NC = 2
NS = 16
C = 64
NCH = 160
E_W = NCH * C
E_PAD = NC * NS * E_W
N_PAD = 10240
RPW = N_PAD // NS
NJ = RPW // C


def _sc_agg_body(dw, *refs):
    (src_hbm, dst_hbm, h_hbm, out_hbm,
     src_i, dst_i, rows, rid, si0, si1, sg0, sg1, acc) = refs

    c = lax.axis_index("c")
    s = lax.axis_index("s")
    wid = c * NS + s
    sem_i = (si0, si1)
    sem_g = (sg0, sg1)

    def idx_copies(k, b):
        return (pltpu.make_async_copy(src_hbm.at[wid, k], src_i.at[b], sem_i[b]),
                pltpu.make_async_copy(dst_hbm.at[wid, k], dst_i.at[b], sem_i[b]))

    def gather_copy(k, b):
        return pltpu.make_async_copy(h_hbm.at[src_i.at[b]], rows.at[b], sem_g[b])

    row0 = s * RPW
    iota16 = lax.iota(jnp.int32, 16)
    for j in range(NJ):
        for v in range(C // 16):
            rid[j, pl.ds(v * 16, 16)] = (row0 + j * C + v * 16) + iota16

    @pl.loop(0, C)
    def _(r):
        for q in range(dw // 16):
            rows[0, r, pl.ds(q * 16, 16)] = jnp.zeros((16,), jnp.float32)

    for j in range(NJ):
        pltpu.sync_copy(rows.at[0], acc.at[rid.at[j]])

    plsc.subcore_barrier()

    @pl.loop(0, NCH)
    def _(k):
        for cp in idx_copies(k, 0):
            cp.start()
        for cp in idx_copies(k, 0):
            cp.wait()
        gather_copy(k, 0).start()
        gather_copy(k, 0).wait()
        pltpu.sync_copy(rows.at[0], acc.at[dst_i.at[0]], add=True)

    plsc.subcore_barrier()

    for j in range(NJ):
        pltpu.sync_copy(acc.at[rid.at[j]], rows.at[0])
        pltpu.sync_copy(rows.at[0], out_hbm.at[c, pl.ds(row0 + j * C, C)])


def _sc_cnt_body(*refs):
    (dst_hbm, out_hbm, dst_i, rows, rid, si0, si1, acc) = refs

    c = lax.axis_index("c")
    s = lax.axis_index("s")
    wid = c * NS + s
    sem_i = (si0, si1)

    def idx_copy(k, b):
        return pltpu.make_async_copy(dst_hbm.at[wid, k], dst_i.at[b], sem_i[b])

    row0 = s * RPW
    iota16 = lax.iota(jnp.int32, 16)
    for j in range(NJ):
        for v in range(C // 16):
            rid[j, pl.ds(v * 16, 16)] = (row0 + j * C + v * 16) + iota16

    @pl.loop(0, C)
    def _(r):
        for q in range(D // 16):
            rows[r, pl.ds(q * 16, 16)] = jnp.zeros((16,), jnp.float32)

    for j in range(NJ):
        pltpu.sync_copy(rows, acc.at[rid.at[j]])

    @pl.loop(0, C)
    def _(r):
        for q in range(D // 16):
            rows[r, pl.ds(q * 16, 16)] = jnp.ones((16,), jnp.float32)

    plsc.subcore_barrier()

    @pl.loop(0, NCH)
    def _(k):
        idx_copy(k, 0).start()
        idx_copy(k, 0).wait()
        pltpu.sync_copy(rows, acc.at[dst_i.at[0]], add=True)

    plsc.subcore_barrier()

    for j in range(NJ):
        pltpu.sync_copy(acc.at[rid.at[j]], rows)
        pltpu.sync_copy(rows, out_hbm.at[c, pl.ds(row0 + j * C, C)])


@functools.lru_cache(maxsize=None)
def _make_sc_cnt():
    mesh = plsc.VectorSubcoreMesh(core_axis_name="c", subcore_axis_name="s")
    scratch = [
        pltpu.VMEM((2, C), jnp.int32),
        pltpu.VMEM((C, D), jnp.float32),
        pltpu.VMEM((NJ, C), jnp.int32),
        pltpu.SemaphoreType.DMA,
        pltpu.SemaphoreType.DMA,
        pltpu.VMEM_SHARED((N_PAD, D), jnp.float32),
    ]
    return pl.kernel(
        _sc_cnt_body,
        out_type=jax.ShapeDtypeStruct((NC, N_PAD, D), jnp.float32),
        mesh=mesh,
        scratch_types=scratch,
    )


@functools.lru_cache(maxsize=None)
def _make_sc_agg(dw):
    mesh = plsc.VectorSubcoreMesh(core_axis_name="c", subcore_axis_name="s")
    scratch = [
        pltpu.VMEM((2, C), jnp.int32),
        pltpu.VMEM((2, C), jnp.int32),
        pltpu.VMEM((2, C, dw), jnp.float32),
        pltpu.VMEM((NJ, C), jnp.int32),
        pltpu.SemaphoreType.DMA,
        pltpu.SemaphoreType.DMA,
        pltpu.SemaphoreType.DMA,
        pltpu.SemaphoreType.DMA,
        pltpu.VMEM_SHARED((N_PAD, dw), jnp.float32),
    ]
    return pl.kernel(
        functools.partial(_sc_agg_body, dw),
        out_type=jax.ShapeDtypeStruct((NC, N_PAD, dw), jnp.float32),
        mesh=mesh,
        scratch_types=scratch,
    )


BR = 2000
G = N // BR


def _dot(a, b):
    return jax.lax.dot_general(a, b, (((1,), (0,)), ((), ())),
                               precision=jax.lax.Precision.HIGHEST,
                               preferred_element_type=jnp.float32)


def _mm0_body(parts_ref, cntp_ref, h_ref, wl_ref, bl_ref, wr_ref,
              t_ref, stat_ref, invc_ref):
    i = pl.program_id(0)
    ssum = parts_ref[0] + parts_ref[1]
    cnt = cntp_ref[0, :, 0:1] + cntp_ref[1, :, 0:1]
    invc = 1.0 / jnp.maximum(cnt, 1.0)
    mean = ssum * invc
    t = _dot(mean, wl_ref[...]) + bl_ref[...] + _dot(h_ref[...], wr_ref[...])
    t_ref[...] = t
    invc_ref[...] = jnp.broadcast_to(invc, (BR, D))

    @pl.when(i == 0)
    def _():
        stat_ref[...] = jnp.zeros((2, D), jnp.float32)

    s1 = jnp.sum(t, axis=0, keepdims=True)
    s2 = jnp.sum(t * t, axis=0, keepdims=True)
    stat_ref[...] += jnp.concatenate([s1, s2], axis=0)


def _mm_body(parts_ref, invc_ref, h_ref, wl_ref, bl_ref, wr_ref,
             t_ref, stat_ref):
    i = pl.program_id(0)
    mean = (parts_ref[0] + parts_ref[1]) * invc_ref[...]
    t = _dot(mean, wl_ref[...]) + bl_ref[...] + _dot(h_ref[...], wr_ref[...])
    t_ref[...] = t

    @pl.when(i == 0)
    def _():
        stat_ref[...] = jnp.zeros((2, D), jnp.float32)

    s1 = jnp.sum(t, axis=0, keepdims=True)
    s2 = jnp.sum(t * t, axis=0, keepdims=True)
    stat_ref[...] += jnp.concatenate([s1, s2], axis=0)


def _bn_body(relu_res, t_ref, h_ref, stat_ref, g_ref, b_ref, o_ref):
    mu = stat_ref[0:1, :] * (1.0 / N)
    var = stat_ref[1:2, :] * (1.0 / N) - mu * mu
    y = (t_ref[...] - mu) * jax.lax.rsqrt(var + EPS) * g_ref[...] + b_ref[...]
    if relu_res:
        y = jnp.maximum(y, 0.0) + h_ref[...]
    o_ref[...] = y


def _bn(t, h, stat, gamma, beta, relu_res):
    return pl.pallas_call(
        functools.partial(_bn_body, relu_res),
        grid=(G,),
        in_specs=[
            pl.BlockSpec((BR, D), lambda i: (i, 0)),
            pl.BlockSpec((BR, D), lambda i: (i, 0)),
            pl.BlockSpec((2, D), lambda i: (0, 0)),
            pl.BlockSpec((D,), lambda i: (0,)),
            pl.BlockSpec((D,), lambda i: (0,)),
        ],
        out_specs=pl.BlockSpec((BR, D), lambda i: (i, 0)),
        out_shape=jax.ShapeDtypeStruct((N, D), jnp.float32),
    )(t, h, stat, gamma, beta)


def _dense0(parts, cntp, h, wl, bl, wr, gamma, beta):
    t, stat, invc = pl.pallas_call(
        _mm0_body,
        grid=(G,),
        in_specs=[
            pl.BlockSpec((NC, BR, D), lambda i: (0, i, 0)),
            pl.BlockSpec((NC, BR, D), lambda i: (0, i, 0)),
            pl.BlockSpec((BR, D), lambda i: (i, 0)),
            pl.BlockSpec((D, D), lambda i: (0, 0)),
            pl.BlockSpec((D,), lambda i: (0,)),
            pl.BlockSpec((D, D), lambda i: (0, 0)),
        ],
        out_specs=[
            pl.BlockSpec((BR, D), lambda i: (i, 0)),
            pl.BlockSpec((2, D), lambda i: (0, 0)),
            pl.BlockSpec((BR, D), lambda i: (i, 0)),
        ],
        out_shape=[
            jax.ShapeDtypeStruct((N, D), jnp.float32),
            jax.ShapeDtypeStruct((2, D), jnp.float32),
            jax.ShapeDtypeStruct((N, D), jnp.float32),
        ],
    )(parts, cntp, h, wl, bl, wr)
    return _bn(t, h, stat, gamma, beta, True), invc


def _dense(parts, invc, h, wl, bl, wr, gamma, beta, relu_res):
    t, stat = pl.pallas_call(
        _mm_body,
        grid=(G,),
        in_specs=[
            pl.BlockSpec((NC, BR, D), lambda i: (0, i, 0)),
            pl.BlockSpec((BR, D), lambda i: (i, 0)),
            pl.BlockSpec((BR, D), lambda i: (i, 0)),
            pl.BlockSpec((D, D), lambda i: (0, 0)),
            pl.BlockSpec((D,), lambda i: (0,)),
            pl.BlockSpec((D, D), lambda i: (0, 0)),
        ],
        out_specs=[
            pl.BlockSpec((BR, D), lambda i: (i, 0)),
            pl.BlockSpec((2, D), lambda i: (0, 0)),
        ],
        out_shape=[
            jax.ShapeDtypeStruct((N, D), jnp.float32),
            jax.ShapeDtypeStruct((2, D), jnp.float32),
        ],
    )(parts, invc, h, wl, bl, wr)
    return _bn(t, h, stat, gamma, beta, relu_res)


def kernel(x, edge_index, Wl0, bl0, Wr0, gamma0, beta0, Wl1, bl1, Wr1,
           gamma1, beta1, Wl2, bl2, Wr2, gamma2, beta2):
    src = edge_index[0]
    dst = edge_index[1]
    pad = E_PAD - E
    src_p = jnp.concatenate([src, jnp.zeros((pad,), jnp.int32)]).reshape(
        NC * NS, NCH, C)
    dst_p = jnp.concatenate([dst, jnp.full((pad,), N, jnp.int32)]).reshape(
        NC * NS, NCH, C)

    cntp = _make_sc_cnt()(dst_p)
    parts0 = _make_sc_agg(D)(src_p, dst_p, x)
    h1, invc = _dense0(parts0, cntp, x, Wl0, bl0, Wr0, gamma0, beta0)
    parts1 = _make_sc_agg(D)(src_p, dst_p, h1)
    h2 = _dense(parts1, invc, h1, Wl1, bl1, Wr1, gamma1, beta1, True)
    parts2 = _make_sc_agg(D)(src_p, dst_p, h2)
    return _dense(parts2, invc, h2, Wl2, bl2, Wr2, gamma2, beta2, False)


# speedup vs baseline: 3.3265x; 1.3445x over previous
import functools

import jax
import jax.numpy as jnp
from jax import lax
from jax.experimental import pallas as pl
from jax.experimental.pallas import tpu as pltpu
from jax.experimental.pallas import tpu_sc as plsc

N = 10000
E = 320000
D = 128
EPS = 1e-5

NC = 2
NS = 16
C = 64
NCH = 160
E_W = NCH * C
E_PAD = NC * NS * E_W
N_PAD = 10240
RPW = N_PAD // NS
NJ = RPW // C


def _sc_agg_body(dw, *refs):
    (src_hbm, dst_hbm, h_hbm, out_hbm,
     src_i, dst_i, rows, rid, si0, si1, sg0, sg1, acc) = refs

    c = lax.axis_index("c")
    s = lax.axis_index("s")
    wid = c * NS + s
    sem_i = (si0, si1)
    sem_g = (sg0, sg1)

    def idx_copies(k, b):
        return (pltpu.make_async_copy(src_hbm.at[wid, k], src_i.at[b], sem_i[b]),
                pltpu.make_async_copy(dst_hbm.at[wid, k], dst_i.at[b], sem_i[b]))

    def gather_copy(k, b):
        return pltpu.make_async_copy(h_hbm.at[src_i.at[b]], rows.at[b], sem_g[b])

    row0 = s * RPW
    iota16 = lax.iota(jnp.int32, 16)
    for j in range(NJ):
        for v in range(C // 16):
            rid[j, pl.ds(v * 16, 16)] = (row0 + j * C + v * 16) + iota16

    @pl.loop(0, C)
    def _(r):
        for q in range(dw // 16):
            rows[0, r, pl.ds(q * 16, 16)] = jnp.zeros((16,), jnp.float32)

    for j in range(NJ):
        pltpu.sync_copy(rows.at[0], acc.at[rid.at[j]])

    plsc.subcore_barrier()

    for cp in idx_copies(0, 0):
        cp.start()
    for cp in idx_copies(1, 1):
        cp.start()
    for cp in idx_copies(0, 0):
        cp.wait()
    gather_copy(0, 0).start()

    @pl.loop(0, NCH, step=2)
    def _(k0):
        for b in range(2):
            k = k0 + b
            o = 1 - b

            @pl.when(k + 1 < NCH)
            def _():
                for cp in idx_copies(k + 1, o):
                    cp.wait()
                gather_copy(k + 1, o).start()

            gather_copy(k, b).wait()
            pltpu.sync_copy(rows.at[b], acc.at[dst_i.at[b]], add=True)

            @pl.when(k + 2 < NCH)
            def _():
                for cp in idx_copies(k + 2, b):
                    cp.start()

    plsc.subcore_barrier()

    for j in range(NJ):
        pltpu.sync_copy(acc.at[rid.at[j]], rows.at[0])
        pltpu.sync_copy(rows.at[0], out_hbm.at[c, pl.ds(row0 + j * C, C)])


def _sc_cnt_body(*refs):
    (dst_hbm, out_hbm, dst_i, rows, rid, si0, si1, acc) = refs

    c = lax.axis_index("c")
    s = lax.axis_index("s")
    wid = c * NS + s
    sem_i = (si0, si1)

    def idx_copy(k, b):
        return pltpu.make_async_copy(dst_hbm.at[wid, k], dst_i.at[b], sem_i[b])

    row0 = s * RPW
    iota16 = lax.iota(jnp.int32, 16)
    for j in range(NJ):
        for v in range(C // 16):
            rid[j, pl.ds(v * 16, 16)] = (row0 + j * C + v * 16) + iota16

    @pl.loop(0, C)
    def _(r):
        for q in range(D // 16):
            rows[r, pl.ds(q * 16, 16)] = jnp.zeros((16,), jnp.float32)

    for j in range(NJ):
        pltpu.sync_copy(rows, acc.at[rid.at[j]])

    @pl.loop(0, C)
    def _(r):
        for q in range(D // 16):
            rows[r, pl.ds(q * 16, 16)] = jnp.ones((16,), jnp.float32)

    plsc.subcore_barrier()

    idx_copy(0, 0).start()
    idx_copy(1, 1).start()

    @pl.loop(0, NCH, step=2)
    def _(k0):
        for b in range(2):
            k = k0 + b
            idx_copy(k, b).wait()
            pltpu.sync_copy(rows, acc.at[dst_i.at[b]], add=True)

            @pl.when(k + 2 < NCH)
            def _():
                idx_copy(k + 2, b).start()

    plsc.subcore_barrier()

    for j in range(NJ):
        pltpu.sync_copy(acc.at[rid.at[j]], rows)
        pltpu.sync_copy(rows, out_hbm.at[c, pl.ds(row0 + j * C, C)])


@functools.lru_cache(maxsize=None)
def _make_sc_cnt():
    mesh = plsc.VectorSubcoreMesh(core_axis_name="c", subcore_axis_name="s")
    scratch = [
        pltpu.VMEM((2, C), jnp.int32),
        pltpu.VMEM((C, D), jnp.float32),
        pltpu.VMEM((NJ, C), jnp.int32),
        pltpu.SemaphoreType.DMA,
        pltpu.SemaphoreType.DMA,
        pltpu.VMEM_SHARED((N_PAD, D), jnp.float32),
    ]
    return pl.kernel(
        _sc_cnt_body,
        out_type=jax.ShapeDtypeStruct((NC, N_PAD, D), jnp.float32),
        mesh=mesh,
        scratch_types=scratch,
    )


@functools.lru_cache(maxsize=None)
def _make_sc_agg(dw):
    mesh = plsc.VectorSubcoreMesh(core_axis_name="c", subcore_axis_name="s")
    scratch = [
        pltpu.VMEM((2, C), jnp.int32),
        pltpu.VMEM((2, C), jnp.int32),
        pltpu.VMEM((2, C, dw), jnp.float32),
        pltpu.VMEM((NJ, C), jnp.int32),
        pltpu.SemaphoreType.DMA,
        pltpu.SemaphoreType.DMA,
        pltpu.SemaphoreType.DMA,
        pltpu.SemaphoreType.DMA,
        pltpu.VMEM_SHARED((N_PAD, dw), jnp.float32),
    ]
    return pl.kernel(
        functools.partial(_sc_agg_body, dw),
        out_type=jax.ShapeDtypeStruct((NC, N_PAD, dw), jnp.float32),
        mesh=mesh,
        scratch_types=scratch,
    )


BR = 2000
G = N // BR


def _dot(a, b):
    return jax.lax.dot_general(a, b, (((1,), (0,)), ((), ())),
                               precision=jax.lax.Precision.HIGHEST,
                               preferred_element_type=jnp.float32)


def _mm0_body(parts_ref, cntp_ref, h_ref, wl_ref, bl_ref, wr_ref,
              t_ref, stat_ref, invc_ref):
    i = pl.program_id(0)
    ssum = parts_ref[0] + parts_ref[1]
    cnt = cntp_ref[0, :, 0:1] + cntp_ref[1, :, 0:1]
    invc = 1.0 / jnp.maximum(cnt, 1.0)
    mean = ssum * invc
    t = _dot(mean, wl_ref[...]) + bl_ref[...] + _dot(h_ref[...], wr_ref[...])
    t_ref[...] = t
    invc_ref[...] = jnp.broadcast_to(invc, (BR, D))

    @pl.when(i == 0)
    def _():
        stat_ref[...] = jnp.zeros((2, D), jnp.float32)

    s1 = jnp.sum(t, axis=0, keepdims=True)
    s2 = jnp.sum(t * t, axis=0, keepdims=True)
    stat_ref[...] += jnp.concatenate([s1, s2], axis=0)


def _mm_body(parts_ref, invc_ref, h_ref, wl_ref, bl_ref, wr_ref,
             t_ref, stat_ref):
    i = pl.program_id(0)
    mean = (parts_ref[0] + parts_ref[1]) * invc_ref[...]
    t = _dot(mean, wl_ref[...]) + bl_ref[...] + _dot(h_ref[...], wr_ref[...])
    t_ref[...] = t

    @pl.when(i == 0)
    def _():
        stat_ref[...] = jnp.zeros((2, D), jnp.float32)

    s1 = jnp.sum(t, axis=0, keepdims=True)
    s2 = jnp.sum(t * t, axis=0, keepdims=True)
    stat_ref[...] += jnp.concatenate([s1, s2], axis=0)


def _bn_body(relu_res, t_ref, h_ref, stat_ref, g_ref, b_ref, o_ref):
    mu = stat_ref[0:1, :] * (1.0 / N)
    var = stat_ref[1:2, :] * (1.0 / N) - mu * mu
    y = (t_ref[...] - mu) * jax.lax.rsqrt(var + EPS) * g_ref[...] + b_ref[...]
    if relu_res:
        y = jnp.maximum(y, 0.0) + h_ref[...]
    o_ref[...] = y


def _bn(t, h, stat, gamma, beta, relu_res):
    return pl.pallas_call(
        functools.partial(_bn_body, relu_res),
        grid=(G,),
        in_specs=[
            pl.BlockSpec((BR, D), lambda i: (i, 0)),
            pl.BlockSpec((BR, D), lambda i: (i, 0)),
            pl.BlockSpec((2, D), lambda i: (0, 0)),
            pl.BlockSpec((D,), lambda i: (0,)),
            pl.BlockSpec((D,), lambda i: (0,)),
        ],
        out_specs=pl.BlockSpec((BR, D), lambda i: (i, 0)),
        out_shape=jax.ShapeDtypeStruct((N, D), jnp.float32),
    )(t, h, stat, gamma, beta)


def _dense0(parts, cntp, h, wl, bl, wr, gamma, beta):
    t, stat, invc = pl.pallas_call(
        _mm0_body,
        grid=(G,),
        in_specs=[
            pl.BlockSpec((NC, BR, D), lambda i: (0, i, 0)),
            pl.BlockSpec((NC, BR, D), lambda i: (0, i, 0)),
            pl.BlockSpec((BR, D), lambda i: (i, 0)),
            pl.BlockSpec((D, D), lambda i: (0, 0)),
            pl.BlockSpec((D,), lambda i: (0,)),
            pl.BlockSpec((D, D), lambda i: (0, 0)),
        ],
        out_specs=[
            pl.BlockSpec((BR, D), lambda i: (i, 0)),
            pl.BlockSpec((2, D), lambda i: (0, 0)),
            pl.BlockSpec((BR, D), lambda i: (i, 0)),
        ],
        out_shape=[
            jax.ShapeDtypeStruct((N, D), jnp.float32),
            jax.ShapeDtypeStruct((2, D), jnp.float32),
            jax.ShapeDtypeStruct((N, D), jnp.float32),
        ],
    )(parts, cntp, h, wl, bl, wr)
    return _bn(t, h, stat, gamma, beta, True), invc


def _dense(parts, invc, h, wl, bl, wr, gamma, beta, relu_res):
    t, stat = pl.pallas_call(
        _mm_body,
        grid=(G,),
        in_specs=[
            pl.BlockSpec((NC, BR, D), lambda i: (0, i, 0)),
            pl.BlockSpec((BR, D), lambda i: (i, 0)),
            pl.BlockSpec((BR, D), lambda i: (i, 0)),
            pl.BlockSpec((D, D), lambda i: (0, 0)),
            pl.BlockSpec((D,), lambda i: (0,)),
            pl.BlockSpec((D, D), lambda i: (0, 0)),
        ],
        out_specs=[
            pl.BlockSpec((BR, D), lambda i: (i, 0)),
            pl.BlockSpec((2, D), lambda i: (0, 0)),
        ],
        out_shape=[
            jax.ShapeDtypeStruct((N, D), jnp.float32),
            jax.ShapeDtypeStruct((2, D), jnp.float32),
        ],
    )(parts, invc, h, wl, bl, wr)
    return _bn(t, h, stat, gamma, beta, relu_res)


def kernel(x, edge_index, Wl0, bl0, Wr0, gamma0, beta0, Wl1, bl1, Wr1,
           gamma1, beta1, Wl2, bl2, Wr2, gamma2, beta2):
    src = edge_index[0]
    dst = edge_index[1]
    pad = E_PAD - E
    src_p = jnp.concatenate([src, jnp.zeros((pad,), jnp.int32)]).reshape(
        NC * NS, NCH, C)
    dst_p = jnp.concatenate([dst, jnp.full((pad,), N, jnp.int32)]).reshape(
        NC * NS, NCH, C)

    cntp = _make_sc_cnt()(dst_p)
    parts0 = _make_sc_agg(D)(src_p, dst_p, x)
    h1, invc = _dense0(parts0, cntp, x, Wl0, bl0, Wr0, gamma0, beta0)
    parts1 = _make_sc_agg(D)(src_p, dst_p, h1)
    h2 = _dense(parts1, invc, h1, Wl1, bl1, Wr1, gamma1, beta1, True)
    parts2 = _make_sc_agg(D)(src_p, dst_p, h2)
    return _dense(parts2, invc, h2, Wl2, bl2, Wr2, gamma2, beta2, False)


# speedup vs baseline: 3.3386x; 1.0037x over previous
import functools

import jax
import jax.numpy as jnp
from jax import lax
from jax.experimental import pallas as pl
from jax.experimental.pallas import tpu as pltpu
from jax.experimental.pallas import tpu_sc as plsc

N = 10000
E = 320000
D = 128
EPS = 1e-5

NC = 2
NS = 16
C = 64
NCH = 160
E_W = NCH * C
E_PAD = NC * NS * E_W
N_PAD = 10240
RPW = N_PAD // NS
NJ = RPW // C


def _sc_agg_body(dw, *refs):
    (src_hbm, dst_hbm, h_hbm, out_hbm,
     src_i, dst_i, rows, rid,
     si0, si1, si2, si3, sg0, sg1, ss0, ss1, acc) = refs

    c = lax.axis_index("c")
    s = lax.axis_index("s")
    wid = c * NS + s
    sem_i = (si0, si1, si2, si3)
    sem_g = (sg0, sg1)
    sem_s = (ss0, ss1)

    def idx_copies(k, q):
        return (pltpu.make_async_copy(src_hbm.at[wid, k], src_i.at[q], sem_i[q]),
                pltpu.make_async_copy(dst_hbm.at[wid, k], dst_i.at[q], sem_i[q]))

    def gather_copy(k, q, b):
        return pltpu.make_async_copy(h_hbm.at[src_i.at[q]], rows.at[b], sem_g[b])

    def scatter_copy(q, b):
        return pltpu.async_copy(rows.at[b], acc.at[dst_i.at[q]], sem_s[b],
                                add=True)

    def scatter_wait(q, b):
        pltpu.make_async_copy(rows.at[b], acc.at[dst_i.at[q]], sem_s[b]).wait()

    row0 = s * RPW
    iota16 = lax.iota(jnp.int32, 16)
    for j in range(NJ):
        for v in range(C // 16):
            rid[j, pl.ds(v * 16, 16)] = (row0 + j * C + v * 16) + iota16

    @pl.loop(0, C)
    def _(r):
        for q in range(dw // 16):
            rows[0, r, pl.ds(q * 16, 16)] = jnp.zeros((16,), jnp.float32)

    for j in range(NJ):
        pltpu.sync_copy(rows.at[0], acc.at[rid.at[j]])

    plsc.subcore_barrier()

    for kk in range(3):
        for cp in idx_copies(kk, kk):
            cp.start()
    for cp in idx_copies(0, 0):
        cp.wait()
    gather_copy(0, 0, 0).start()

    @pl.loop(0, NCH, step=4)
    def _(k0):
        for b4 in range(4):
            k = k0 + b4
            b = b4 % 2
            o = 1 - b
            q = b4
            qn = (b4 + 1) % 4

            @pl.when(k + 1 < NCH)
            def _():
                for cp in idx_copies(k + 1, qn):
                    cp.wait()

            @pl.when((k + 1 < NCH) & (k >= 1))
            def _():
                scatter_wait((b4 + 3) % 4, o)

            @pl.when(k + 1 < NCH)
            def _():
                gather_copy(k + 1, qn, o).start()

            gather_copy(k, q, b).wait()
            scatter_copy(q, b)

            @pl.when(k + 3 < NCH)
            def _():
                for cp in idx_copies(k + 3, (b4 + 3) % 4):
                    cp.start()

    scatter_wait((NCH - 2) % 4, (NCH - 2) % 2)
    scatter_wait((NCH - 1) % 4, (NCH - 1) % 2)
    plsc.subcore_barrier()

    for j in range(NJ):
        pltpu.sync_copy(acc.at[rid.at[j]], rows.at[0])
        pltpu.sync_copy(rows.at[0], out_hbm.at[c, pl.ds(row0 + j * C, C)])


def _sc_cnt_body(*refs):
    (dst_hbm, out_hbm, dst_i, rows, rid, si0, si1, acc) = refs

    c = lax.axis_index("c")
    s = lax.axis_index("s")
    wid = c * NS + s
    sem_i = (si0, si1)

    def idx_copy(k, b):
        return pltpu.make_async_copy(dst_hbm.at[wid, k], dst_i.at[b], sem_i[b])

    row0 = s * RPW
    iota16 = lax.iota(jnp.int32, 16)
    for j in range(NJ):
        for v in range(C // 16):
            rid[j, pl.ds(v * 16, 16)] = (row0 + j * C + v * 16) + iota16

    @pl.loop(0, C)
    def _(r):
        for q in range(D // 16):
            rows[r, pl.ds(q * 16, 16)] = jnp.zeros((16,), jnp.float32)

    for j in range(NJ):
        pltpu.sync_copy(rows, acc.at[rid.at[j]])

    @pl.loop(0, C)
    def _(r):
        for q in range(D // 16):
            rows[r, pl.ds(q * 16, 16)] = jnp.ones((16,), jnp.float32)

    plsc.subcore_barrier()

    idx_copy(0, 0).start()
    idx_copy(1, 1).start()

    @pl.loop(0, NCH, step=2)
    def _(k0):
        for b in range(2):
            k = k0 + b
            idx_copy(k, b).wait()
            pltpu.sync_copy(rows, acc.at[dst_i.at[b]], add=True)

            @pl.when(k + 2 < NCH)
            def _():
                idx_copy(k + 2, b).start()

    plsc.subcore_barrier()

    for j in range(NJ):
        pltpu.sync_copy(acc.at[rid.at[j]], rows)
        pltpu.sync_copy(rows, out_hbm.at[c, pl.ds(row0 + j * C, C)])


@functools.lru_cache(maxsize=None)
def _make_sc_cnt():
    mesh = plsc.VectorSubcoreMesh(core_axis_name="c", subcore_axis_name="s")
    scratch = [
        pltpu.VMEM((2, C), jnp.int32),
        pltpu.VMEM((C, D), jnp.float32),
        pltpu.VMEM((NJ, C), jnp.int32),
        pltpu.SemaphoreType.DMA,
        pltpu.SemaphoreType.DMA,
        pltpu.VMEM_SHARED((N_PAD, D), jnp.float32),
    ]
    return pl.kernel(
        _sc_cnt_body,
        out_type=jax.ShapeDtypeStruct((NC, N_PAD, D), jnp.float32),
        mesh=mesh,
        scratch_types=scratch,
    )


@functools.lru_cache(maxsize=None)
def _make_sc_agg(dw):
    mesh = plsc.VectorSubcoreMesh(core_axis_name="c", subcore_axis_name="s")
    scratch = [
        pltpu.VMEM((4, C), jnp.int32),
        pltpu.VMEM((4, C), jnp.int32),
        pltpu.VMEM((2, C, dw), jnp.float32),
        pltpu.VMEM((NJ, C), jnp.int32),
        pltpu.SemaphoreType.DMA,
        pltpu.SemaphoreType.DMA,
        pltpu.SemaphoreType.DMA,
        pltpu.SemaphoreType.DMA,
        pltpu.SemaphoreType.DMA,
        pltpu.SemaphoreType.DMA,
        pltpu.SemaphoreType.DMA,
        pltpu.SemaphoreType.DMA,
        pltpu.VMEM_SHARED((N_PAD, dw), jnp.float32),
    ]
    return pl.kernel(
        functools.partial(_sc_agg_body, dw),
        out_type=jax.ShapeDtypeStruct((NC, N_PAD, dw), jnp.float32),
        mesh=mesh,
        scratch_types=scratch,
    )


BR = 2000
G = N // BR


def _dot(a, b):
    return jax.lax.dot_general(a, b, (((1,), (0,)), ((), ())),
                               precision=jax.lax.Precision.HIGHEST,
                               preferred_element_type=jnp.float32)


def _mm0_body(parts_ref, cntp_ref, h_ref, wl_ref, bl_ref, wr_ref,
              t_ref, stat_ref, invc_ref):
    i = pl.program_id(0)
    ssum = parts_ref[0] + parts_ref[1]
    cnt = cntp_ref[0, :, 0:1] + cntp_ref[1, :, 0:1]
    invc = 1.0 / jnp.maximum(cnt, 1.0)
    mean = ssum * invc
    t = _dot(mean, wl_ref[...]) + bl_ref[...] + _dot(h_ref[...], wr_ref[...])
    t_ref[...] = t
    invc_ref[...] = jnp.broadcast_to(invc, (BR, D))

    @pl.when(i == 0)
    def _():
        stat_ref[...] = jnp.zeros((2, D), jnp.float32)

    s1 = jnp.sum(t, axis=0, keepdims=True)
    s2 = jnp.sum(t * t, axis=0, keepdims=True)
    stat_ref[...] += jnp.concatenate([s1, s2], axis=0)


def _mm_body(parts_ref, invc_ref, h_ref, wl_ref, bl_ref, wr_ref,
             t_ref, stat_ref):
    i = pl.program_id(0)
    mean = (parts_ref[0] + parts_ref[1]) * invc_ref[...]
    t = _dot(mean, wl_ref[...]) + bl_ref[...] + _dot(h_ref[...], wr_ref[...])
    t_ref[...] = t

    @pl.when(i == 0)
    def _():
        stat_ref[...] = jnp.zeros((2, D), jnp.float32)

    s1 = jnp.sum(t, axis=0, keepdims=True)
    s2 = jnp.sum(t * t, axis=0, keepdims=True)
    stat_ref[...] += jnp.concatenate([s1, s2], axis=0)


def _bn_body(relu_res, t_ref, h_ref, stat_ref, g_ref, b_ref, o_ref):
    mu = stat_ref[0:1, :] * (1.0 / N)
    var = stat_ref[1:2, :] * (1.0 / N) - mu * mu
    y = (t_ref[...] - mu) * jax.lax.rsqrt(var + EPS) * g_ref[...] + b_ref[...]
    if relu_res:
        y = jnp.maximum(y, 0.0) + h_ref[...]
    o_ref[...] = y


def _bn(t, h, stat, gamma, beta, relu_res):
    return pl.pallas_call(
        functools.partial(_bn_body, relu_res),
        grid=(G,),
        in_specs=[
            pl.BlockSpec((BR, D), lambda i: (i, 0)),
            pl.BlockSpec((BR, D), lambda i: (i, 0)),
            pl.BlockSpec((2, D), lambda i: (0, 0)),
            pl.BlockSpec((D,), lambda i: (0,)),
            pl.BlockSpec((D,), lambda i: (0,)),
        ],
        out_specs=pl.BlockSpec((BR, D), lambda i: (i, 0)),
        out_shape=jax.ShapeDtypeStruct((N, D), jnp.float32),
    )(t, h, stat, gamma, beta)


def _dense0(parts, cntp, h, wl, bl, wr, gamma, beta):
    t, stat, invc = pl.pallas_call(
        _mm0_body,
        grid=(G,),
        in_specs=[
            pl.BlockSpec((NC, BR, D), lambda i: (0, i, 0)),
            pl.BlockSpec((NC, BR, D), lambda i: (0, i, 0)),
            pl.BlockSpec((BR, D), lambda i: (i, 0)),
            pl.BlockSpec((D, D), lambda i: (0, 0)),
            pl.BlockSpec((D,), lambda i: (0,)),
            pl.BlockSpec((D, D), lambda i: (0, 0)),
        ],
        out_specs=[
            pl.BlockSpec((BR, D), lambda i: (i, 0)),
            pl.BlockSpec((2, D), lambda i: (0, 0)),
            pl.BlockSpec((BR, D), lambda i: (i, 0)),
        ],
        out_shape=[
            jax.ShapeDtypeStruct((N, D), jnp.float32),
            jax.ShapeDtypeStruct((2, D), jnp.float32),
            jax.ShapeDtypeStruct((N, D), jnp.float32),
        ],
    )(parts, cntp, h, wl, bl, wr)
    return _bn(t, h, stat, gamma, beta, True), invc


def _dense(parts, invc, h, wl, bl, wr, gamma, beta, relu_res):
    t, stat = pl.pallas_call(
        _mm_body,
        grid=(G,),
        in_specs=[
            pl.BlockSpec((NC, BR, D), lambda i: (0, i, 0)),
            pl.BlockSpec((BR, D), lambda i: (i, 0)),
            pl.BlockSpec((BR, D), lambda i: (i, 0)),
            pl.BlockSpec((D, D), lambda i: (0, 0)),
            pl.BlockSpec((D,), lambda i: (0,)),
            pl.BlockSpec((D, D), lambda i: (0, 0)),
        ],
        out_specs=[
            pl.BlockSpec((BR, D), lambda i: (i, 0)),
            pl.BlockSpec((2, D), lambda i: (0, 0)),
        ],
        out_shape=[
            jax.ShapeDtypeStruct((N, D), jnp.float32),
            jax.ShapeDtypeStruct((2, D), jnp.float32),
        ],
    )(parts, invc, h, wl, bl, wr)
    return _bn(t, h, stat, gamma, beta, relu_res)


def kernel(x, edge_index, Wl0, bl0, Wr0, gamma0, beta0, Wl1, bl1, Wr1,
           gamma1, beta1, Wl2, bl2, Wr2, gamma2, beta2):
    src = edge_index[0]
    dst = edge_index[1]
    pad = E_PAD - E
    src_p = jnp.concatenate([src, jnp.zeros((pad,), jnp.int32)]).reshape(
        NC * NS, NCH, C)
    dst_p = jnp.concatenate([dst, jnp.full((pad,), N, jnp.int32)]).reshape(
        NC * NS, NCH, C)

    cntp = _make_sc_cnt()(dst_p)
    parts0 = _make_sc_agg(D)(src_p, dst_p, x)
    h1, invc = _dense0(parts0, cntp, x, Wl0, bl0, Wr0, gamma0, beta0)
    parts1 = _make_sc_agg(D)(src_p, dst_p, h1)
    h2 = _dense(parts1, invc, h1, Wl1, bl1, Wr1, gamma1, beta1, True)
    parts2 = _make_sc_agg(D)(src_p, dst_p, h2)
    return _dense(parts2, invc, h2, Wl2, bl2, Wr2, gamma2, beta2, False)


# speedup vs baseline: 9.4613x; 2.8339x over previous
import functools

import jax
import jax.numpy as jnp
from jax import lax
from jax.experimental import pallas as pl
from jax.experimental.pallas import tpu as pltpu
from jax.experimental.pallas import tpu_sc as plsc

N = 10000
E = 320000
D = 128
EPS = 1e-5

NC = 2
NS = 16
C = 64
NCH = 160
E_W = NCH * C
E_PAD = NC * NS * E_W
N_PAD = 10240
RPW = N_PAD // NS
NJ = RPW // C


def _sc_agg_body(dw, *refs):
    (src_hbm, dst_hbm, h_hbm, out_hbm,
     src_i, dst_i, rows, rid,
     si0, si1, si2, si3, sg0, sg1, ss0, ss1, acc) = refs

    c = lax.axis_index("c")
    s = lax.axis_index("s")
    wid = c * NS + s
    sem_i = (si0, si1, si2, si3)
    sem_g = (sg0, sg1)
    sem_s = (ss0, ss1)

    def idx_copies(k, q):
        return (pltpu.make_async_copy(src_hbm.at[wid, k], src_i.at[q], sem_i[q]),
                pltpu.make_async_copy(dst_hbm.at[wid, k], dst_i.at[q], sem_i[q]))

    def gather_copy(k, q, b):
        return pltpu.make_async_copy(h_hbm.at[src_i.at[q]], rows.at[b], sem_g[b])

    def scatter_copy(q, b):
        return pltpu.async_copy(rows.at[b], acc.at[dst_i.at[q]], sem_s[b],
                                add=True)

    def scatter_wait(q, b):
        pltpu.make_async_copy(rows.at[b], acc.at[dst_i.at[q]], sem_s[b]).wait()

    row0 = s * RPW
    iota16 = lax.iota(jnp.int32, 16)
    for j in range(NJ):
        for v in range(C // 16):
            rid[j, pl.ds(v * 16, 16)] = (row0 + j * C + v * 16) + iota16

    @pl.loop(0, C)
    def _(r):
        for q in range(dw // 16):
            rows[0, r, pl.ds(q * 16, 16)] = jnp.zeros((16,), jnp.float32)

    for j in range(NJ):
        pltpu.sync_copy(rows.at[0], acc.at[rid.at[j]])

    plsc.subcore_barrier()

    for kk in range(3):
        for cp in idx_copies(kk, kk):
            cp.start()
    for cp in idx_copies(0, 0):
        cp.wait()
    gather_copy(0, 0, 0).start()

    @pl.loop(0, NCH, step=4)
    def _(k0):
        for b4 in range(4):
            k = k0 + b4
            b = b4 % 2
            o = 1 - b
            q = b4
            qn = (b4 + 1) % 4

            @pl.when(k + 1 < NCH)
            def _():
                for cp in idx_copies(k + 1, qn):
                    cp.wait()

            @pl.when((k + 1 < NCH) & (k >= 1))
            def _():
                scatter_wait((b4 + 3) % 4, o)

            @pl.when(k + 1 < NCH)
            def _():
                gather_copy(k + 1, qn, o).start()

            gather_copy(k, q, b).wait()
            scatter_copy(q, b)

            @pl.when(k + 3 < NCH)
            def _():
                for cp in idx_copies(k + 3, (b4 + 3) % 4):
                    cp.start()

    scatter_wait((NCH - 2) % 4, (NCH - 2) % 2)
    scatter_wait((NCH - 1) % 4, (NCH - 1) % 2)
    plsc.subcore_barrier()

    for j in range(NJ):
        pltpu.sync_copy(acc.at[rid.at[j]], rows.at[0])
        pltpu.sync_copy(rows.at[0], out_hbm.at[c, pl.ds(row0 + j * C, C)])


def _sc_cnt_body(*refs):
    (dst_hbm, out_hbm, dst_i, rows, rid, si0, si1, acc) = refs

    c = lax.axis_index("c")
    s = lax.axis_index("s")
    wid = c * NS + s
    sem_i = (si0, si1)

    def idx_copy(k, b):
        return pltpu.make_async_copy(dst_hbm.at[wid, k], dst_i.at[b], sem_i[b])

    row0 = s * RPW
    iota16 = lax.iota(jnp.int32, 16)
    for j in range(NJ):
        for v in range(C // 16):
            rid[j, pl.ds(v * 16, 16)] = (row0 + j * C + v * 16) + iota16

    @pl.loop(0, C)
    def _(r):
        for q in range(D // 16):
            rows[r, pl.ds(q * 16, 16)] = jnp.zeros((16,), jnp.float32)

    for j in range(NJ):
        pltpu.sync_copy(rows, acc.at[rid.at[j]])

    @pl.loop(0, C)
    def _(r):
        for q in range(D // 16):
            rows[r, pl.ds(q * 16, 16)] = jnp.ones((16,), jnp.float32)

    plsc.subcore_barrier()

    idx_copy(0, 0).start()
    idx_copy(1, 1).start()

    @pl.loop(0, NCH, step=2)
    def _(k0):
        for b in range(2):
            k = k0 + b
            idx_copy(k, b).wait()
            pltpu.sync_copy(rows, acc.at[dst_i.at[b]], add=True)

            @pl.when(k + 2 < NCH)
            def _():
                idx_copy(k + 2, b).start()

    plsc.subcore_barrier()

    for j in range(NJ):
        pltpu.sync_copy(acc.at[rid.at[j]], rows)
        pltpu.sync_copy(rows, out_hbm.at[c, pl.ds(row0 + j * C, C)])


@functools.lru_cache(maxsize=None)
def _make_sc_cnt():
    mesh = plsc.VectorSubcoreMesh(core_axis_name="c", subcore_axis_name="s")
    scratch = [
        pltpu.VMEM((2, C), jnp.int32),
        pltpu.VMEM((C, D), jnp.float32),
        pltpu.VMEM((NJ, C), jnp.int32),
        pltpu.SemaphoreType.DMA,
        pltpu.SemaphoreType.DMA,
        pltpu.VMEM_SHARED((N_PAD, D), jnp.float32),
    ]
    return pl.kernel(
        _sc_cnt_body,
        out_type=jax.ShapeDtypeStruct((NC, N_PAD, D), jnp.float32),
        mesh=mesh,
        scratch_types=scratch,
    )


@functools.lru_cache(maxsize=None)
def _make_sc_agg(dw):
    mesh = plsc.VectorSubcoreMesh(core_axis_name="c", subcore_axis_name="s")
    scratch = [
        pltpu.VMEM((4, C), jnp.int32),
        pltpu.VMEM((4, C), jnp.int32),
        pltpu.VMEM((2, C, dw), jnp.float32),
        pltpu.VMEM((NJ, C), jnp.int32),
        pltpu.SemaphoreType.DMA,
        pltpu.SemaphoreType.DMA,
        pltpu.SemaphoreType.DMA,
        pltpu.SemaphoreType.DMA,
        pltpu.SemaphoreType.DMA,
        pltpu.SemaphoreType.DMA,
        pltpu.SemaphoreType.DMA,
        pltpu.SemaphoreType.DMA,
        pltpu.VMEM_SHARED((N_PAD, dw), jnp.float32),
    ]
    return pl.kernel(
        functools.partial(_sc_agg_body, dw),
        out_type=jax.ShapeDtypeStruct((NC, N_PAD, dw), jnp.float32),
        mesh=mesh,
        scratch_types=scratch,
    )


BR = 2000
G = N // BR


def _dot(a, b):
    return jax.lax.dot_general(a, b, (((1,), (0,)), ((), ())),
                               precision=jax.lax.Precision.HIGHEST,
                               preferred_element_type=jnp.float32)


def _mm0_body(parts_ref, cntp_ref, h_ref, wl_ref, bl_ref, wr_ref,
              t_ref, stat_ref, invc_ref):
    i = pl.program_id(0)
    ssum = parts_ref[0] + parts_ref[1]
    cnt = cntp_ref[0, :, 0:1] + cntp_ref[1, :, 0:1]
    invc = 1.0 / jnp.maximum(cnt, 1.0)
    mean = ssum * invc
    t = _dot(mean, wl_ref[...]) + bl_ref[...] + _dot(h_ref[...], wr_ref[...])
    t_ref[...] = t
    invc_ref[...] = jnp.broadcast_to(invc, (BR, D))

    @pl.when(i == 0)
    def _():
        stat_ref[...] = jnp.zeros((2, D), jnp.float32)

    s1 = jnp.sum(t, axis=0, keepdims=True)
    s2 = jnp.sum(t * t, axis=0, keepdims=True)
    stat_ref[...] += jnp.concatenate([s1, s2], axis=0)


def _mm_body(parts_ref, invc_ref, h_ref, wl_ref, bl_ref, wr_ref,
             t_ref, stat_ref):
    i = pl.program_id(0)
    mean = (parts_ref[0] + parts_ref[1]) * invc_ref[...]
    t = _dot(mean, wl_ref[...]) + bl_ref[...] + _dot(h_ref[...], wr_ref[...])
    t_ref[...] = t

    @pl.when(i == 0)
    def _():
        stat_ref[...] = jnp.zeros((2, D), jnp.float32)

    s1 = jnp.sum(t, axis=0, keepdims=True)
    s2 = jnp.sum(t * t, axis=0, keepdims=True)
    stat_ref[...] += jnp.concatenate([s1, s2], axis=0)


def _bn_body(relu_res, t_ref, h_ref, stat_ref, g_ref, b_ref, o_ref):
    mu = stat_ref[0:1, :] * (1.0 / N)
    var = stat_ref[1:2, :] * (1.0 / N) - mu * mu
    y = (t_ref[...] - mu) * jax.lax.rsqrt(var + EPS) * g_ref[...] + b_ref[...]
    if relu_res:
        y = jnp.maximum(y, 0.0) + h_ref[...]
    o_ref[...] = y


def _bn(t, h, stat, gamma, beta, relu_res):
    return pl.pallas_call(
        functools.partial(_bn_body, relu_res),
        grid=(G,),
        in_specs=[
            pl.BlockSpec((BR, D), lambda i: (i, 0)),
            pl.BlockSpec((BR, D), lambda i: (i, 0)),
            pl.BlockSpec((2, D), lambda i: (0, 0)),
            pl.BlockSpec((D,), lambda i: (0,)),
            pl.BlockSpec((D,), lambda i: (0,)),
        ],
        out_specs=pl.BlockSpec((BR, D), lambda i: (i, 0)),
        out_shape=jax.ShapeDtypeStruct((N, D), jnp.float32),
    )(t, h, stat, gamma, beta)


def _dense0(parts, cntp, h, wl, bl, wr, gamma, beta):
    t, stat, invc = pl.pallas_call(
        _mm0_body,
        grid=(G,),
        in_specs=[
            pl.BlockSpec((NC, BR, D), lambda i: (0, i, 0)),
            pl.BlockSpec((NC, BR, D), lambda i: (0, i, 0)),
            pl.BlockSpec((BR, D), lambda i: (i, 0)),
            pl.BlockSpec((D, D), lambda i: (0, 0)),
            pl.BlockSpec((D,), lambda i: (0,)),
            pl.BlockSpec((D, D), lambda i: (0, 0)),
        ],
        out_specs=[
            pl.BlockSpec((BR, D), lambda i: (i, 0)),
            pl.BlockSpec((2, D), lambda i: (0, 0)),
            pl.BlockSpec((BR, D), lambda i: (i, 0)),
        ],
        out_shape=[
            jax.ShapeDtypeStruct((N, D), jnp.float32),
            jax.ShapeDtypeStruct((2, D), jnp.float32),
            jax.ShapeDtypeStruct((N, D), jnp.float32),
        ],
    )(parts, cntp, h, wl, bl, wr)
    return _bn(t, h, stat, gamma, beta, True), invc


def _dense(parts, invc, h, wl, bl, wr, gamma, beta, relu_res):
    t, stat = pl.pallas_call(
        _mm_body,
        grid=(G,),
        in_specs=[
            pl.BlockSpec((NC, BR, D), lambda i: (0, i, 0)),
            pl.BlockSpec((BR, D), lambda i: (i, 0)),
            pl.BlockSpec((BR, D), lambda i: (i, 0)),
            pl.BlockSpec((D, D), lambda i: (0, 0)),
            pl.BlockSpec((D,), lambda i: (0,)),
            pl.BlockSpec((D, D), lambda i: (0, 0)),
        ],
        out_specs=[
            pl.BlockSpec((BR, D), lambda i: (i, 0)),
            pl.BlockSpec((2, D), lambda i: (0, 0)),
        ],
        out_shape=[
            jax.ShapeDtypeStruct((N, D), jnp.float32),
            jax.ShapeDtypeStruct((2, D), jnp.float32),
        ],
    )(parts, invc, h, wl, bl, wr)
    return _bn(t, h, stat, gamma, beta, relu_res)


def kernel(x, edge_index, Wl0, bl0, Wr0, gamma0, beta0, Wl1, bl1, Wr1,
           gamma1, beta1, Wl2, bl2, Wr2, gamma2, beta2):
    src = edge_index[0]
    dst = edge_index[1]
    pad = E_PAD - E
    ar = jnp.arange(pad, dtype=jnp.int32)
    src_p = jnp.concatenate([src, ar % N]).reshape(NC * NS, NCH, C)
    dst_p = jnp.concatenate([dst, N + ar % (N_PAD - N)]).reshape(
        NC * NS, NCH, C)

    cntp = _make_sc_cnt()(dst_p)
    parts0 = _make_sc_agg(D)(src_p, dst_p, x)
    h1, invc = _dense0(parts0, cntp, x, Wl0, bl0, Wr0, gamma0, beta0)
    parts1 = _make_sc_agg(D)(src_p, dst_p, h1)
    h2 = _dense(parts1, invc, h1, Wl1, bl1, Wr1, gamma1, beta1, True)
    parts2 = _make_sc_agg(D)(src_p, dst_p, h2)
    return _dense(parts2, invc, h2, Wl2, bl2, Wr2, gamma2, beta2, False)
